# baseline (speedup 1.0000x reference)
import jax
import jax.numpy as jnp
from jax import lax
from jax.experimental import pallas as pl
from jax.experimental.pallas import tpu as pltpu

E = 8
K = 2
H = 1024
F = 1408
T = 2048

TM = 256
FC = 128
NF = F // FC

_NT = (((1,), (1,)), ((), ()))


def _moe_dense_kernel(ids_ref, w_ref, x_ref, gu_ref, dn_ref, out_ref, act_ref):
    e = pl.program_id(0)
    f = pl.program_id(1)

    guc = gu_ref[0].astype(jnp.bfloat16)
    gate_w = guc[0]
    up_w = guc[1]

    fcols = pl.ds(f * FC, FC)
    for t in range(T // TM):
        tok = pl.ds(t * TM, TM)
        x_tile = x_ref[tok, :]
        hg = lax.dot_general(x_tile, gate_w, _NT,
                             preferred_element_type=jnp.float32)
        hu = lax.dot_general(x_tile, up_w, _NT,
                             preferred_element_type=jnp.float32)
        act_ref[tok, fcols] = (jax.nn.silu(hg) * hu).astype(jnp.bfloat16)

    @pl.when(f == NF - 1)
    def _down():
        dnc = dn_ref[0].astype(jnp.bfloat16)
        for t in range(T // TM):
            tok = pl.ds(t * TM, TM)
            y = lax.dot_general(act_ref[tok, :], dnc, _NT,
                                preferred_element_type=jnp.float32)
            ids = ids_ref[tok, :]
            w = w_ref[tok, :]
            wte = jnp.sum(jnp.where(ids == e, w, 0.0), axis=1)
            contrib = y * wte[:, None]

            @pl.when(e == 0)
            def _init():
                out_ref[tok, :] = contrib

            @pl.when(e > 0)
            def _acc():
                out_ref[tok, :] += contrib


@jax.jit
def kernel(x, topk_ids, topk_weight, gate_up_weights, down_weights):
    ids = topk_ids.astype(jnp.int32)
    xb = x.astype(jnp.bfloat16)
    gu4 = gate_up_weights.reshape(E, 2, F, H)

    grid = (E, NF)
    out = pl.pallas_call(
        _moe_dense_kernel,
        grid=grid,
        in_specs=[
            pl.BlockSpec((T, K), lambda e, f: (0, 0)),
            pl.BlockSpec((T, K), lambda e, f: (0, 0)),
            pl.BlockSpec((T, H), lambda e, f: (0, 0)),
            pl.BlockSpec((1, 2, FC, H), lambda e, f: (e, 0, f, 0)),
            pl.BlockSpec((1, H, F), lambda e, f: (e, 0, 0)),
        ],
        out_specs=pl.BlockSpec((T, H), lambda e, f: (0, 0)),
        out_shape=jax.ShapeDtypeStruct((T, H), jnp.float32),
        scratch_shapes=[pltpu.VMEM((T, F), jnp.bfloat16)],
    )(ids, topk_weight, xb, gu4, down_weights)
    return out
